# Initial kernel scaffold; baseline (speedup 1.0000x reference)
import jax
import jax.numpy as jnp
from jax.experimental import pallas as pl


def kernel(x, edge_index, W1, b1, W2, b2):
    raise NotImplementedError("write your pallas kernel here")


# speedup vs baseline: 16.3989x; 16.3989x over previous
import functools

import jax
import jax.numpy as jnp
from jax import lax
from jax.experimental import pallas as pl
from jax.experimental.pallas import tpu as pltpu
from jax.experimental.pallas import tpu_sc as plsc

N = 10000
E = 320000
F_IN = 128
NH1 = 64
NCLS = 10
K = 8
H2 = 16

NC = 2
NS = 16
NW = NC * NS

C = 80
ROWS_ALL = E // C
ROWS_PW = ROWS_ALL // NW
RPT = N // NS

_MESH = plsc.VectorSubcoreMesh(core_axis_name="c", subcore_axis_name="s")


RA = 632
RB = N - (NS - 1) * RA

_SC_PARAMS = pltpu.CompilerParams(use_tc_tiling_on_sc=False)


def _make_sc_degree():
    @functools.partial(
        pl.kernel,
        out_type=jax.ShapeDtypeStruct((NC * N,), jnp.float32),
        mesh=_MESH,
        compiler_params=_SC_PARAMS,
        scratch_types=[
            pltpu.VMEM((ROWS_PW, C), jnp.int32),
            pltpu.VMEM((C,), jnp.float32),
            pltpu.VMEM_SHARED((N,), jnp.float32),
        ],
    )
    def degree(srcr, zeros1, out, src_v, ones_v, dacc):
        cid = lax.axis_index("c")
        sid = lax.axis_index("s")
        wid = cid * NS + sid
        start = sid * RA

        @pl.loop(0, C, step=16)
        def _(j):
            ones_v[pl.ds(j, 16)] = jnp.ones((16,), jnp.float32)

        @pl.when(sid < NS - 1)
        def _():
            pltpu.sync_copy(zeros1.at[pl.ds(start, RA)], dacc.at[pl.ds(start, RA)])

        @pl.when(sid == NS - 1)
        def _():
            pltpu.sync_copy(zeros1.at[pl.ds(start, RB)], dacc.at[pl.ds(start, RB)])

        pltpu.sync_copy(srcr.at[wid], src_v)
        plsc.subcore_barrier()

        @pl.loop(0, ROWS_PW)
        def _(i):
            pltpu.sync_copy(ones_v, dacc.at[src_v.at[i]], add=True)

        plsc.subcore_barrier()

        @pl.when(sid < NS - 1)
        def _():
            pltpu.sync_copy(dacc.at[pl.ds(start, RA)],
                            out.at[pl.ds(cid * N + start, RA)])

        @pl.when(sid == NS - 1)
        def _():
            pltpu.sync_copy(dacc.at[pl.ds(start, RB)],
                            out.at[pl.ds(cid * N + start, RB)])

    return degree


def _make_sc_prop(H):
    @functools.partial(
        pl.kernel,
        out_type=jax.ShapeDtypeStruct((NC, N, H), jnp.float32),
        mesh=_MESH,
        compiler_params=_SC_PARAMS,
        scratch_types=[
            pltpu.VMEM((ROWS_PW, C), jnp.int32),
            pltpu.VMEM((ROWS_PW, C), jnp.int32),
            pltpu.VMEM((C, H), jnp.float32),
            pltpu.VMEM((C, H), jnp.float32),
            pltpu.VMEM_SHARED((N, H), jnp.float32),
            pltpu.SemaphoreType.DMA,
            pltpu.SemaphoreType.DMA,
        ],
    )
    def prop(beta, srcr, dstr, zeros2, out, src_v, dst_v, rows0, rows1, acc, s0, s1):
        cid = lax.axis_index("c")
        sid = lax.axis_index("s")
        wid = cid * NS + sid
        start = sid * RA

        @pl.when(sid < NS - 1)
        def _():
            pltpu.sync_copy(zeros2.at[pl.ds(start, RA)], acc.at[pl.ds(start, RA)])

        @pl.when(sid == NS - 1)
        def _():
            pltpu.sync_copy(zeros2.at[pl.ds(start, RB)], acc.at[pl.ds(start, RB)])

        pltpu.sync_copy(srcr.at[wid], src_v)
        pltpu.sync_copy(dstr.at[wid], dst_v)
        plsc.subcore_barrier()

        def gather(i, buf, sem):
            return pltpu.make_async_copy(beta.at[src_v.at[i]], buf, sem)

        def scatter(i, buf):
            pltpu.sync_copy(buf, acc.at[dst_v.at[i]], add=True)

        gather(0, rows0, s0).start()

        @pl.loop(0, ROWS_PW - 1, step=2)
        def _(i):
            gather(i + 1, rows1, s1).start()
            gather(i, rows0, s0).wait()
            scatter(i, rows0)
            gather(i + 2, rows0, s0).start()
            gather(i + 1, rows1, s1).wait()
            scatter(i + 1, rows1)

        gather(ROWS_PW - 1, rows0, s0).wait()
        scatter(ROWS_PW - 1, rows0)

        plsc.subcore_barrier()

        @pl.when(sid < NS - 1)
        def _():
            pltpu.sync_copy(acc.at[pl.ds(start, RA)],
                            out.at[cid, pl.ds(start, RA)])

        @pl.when(sid == NS - 1)
        def _():
            pltpu.sync_copy(acc.at[pl.ds(start, RB)],
                            out.at[cid, pl.ds(start, RB)])

    return prop


def _tc_precompute(deg0, deg1):
    def body(d0_ref, d1_ref, dinv_ref, g_ref, dsq_ref):
        deg = d0_ref[...] + d1_ref[...]
        pos = deg > 0
        safe = jnp.where(pos, deg, 1.0)
        dinv_ref[...] = jnp.where(pos, lax.rsqrt(safe), 0.0)
        g_ref[...] = jnp.where(pos, 1.0 / safe, 0.0)
        dsq_ref[...] = jnp.where(pos, jnp.sqrt(safe), 0.0)

    shp = jax.ShapeDtypeStruct((N, 1), jnp.float32)
    return pl.pallas_call(
        body,
        grid=(1,),
        in_specs=[pl.BlockSpec((N, 1), lambda i: (0, 0)),
                  pl.BlockSpec((N, 1), lambda i: (0, 0))],
        out_specs=[pl.BlockSpec((N, 1), lambda i: (0, 0))] * 3,
        out_shape=[shp, shp, shp],
    )(deg0, deg1)


def _tc_project(x, wcat, dinv, H):
    F = x.shape[1]
    M = wcat.shape[1]
    NB = 1000

    def body(x_ref, w_ref, d_ref, o_ref):
        u = jnp.dot(x_ref[...], w_ref[...], preferred_element_type=jnp.float32)
        col = lax.broadcasted_iota(jnp.int32, (NB, M), 1)
        m = (col >= H) & (col < 8 * H)
        o_ref[...] = jnp.where(m, u * d_ref[...], u)

    return pl.pallas_call(
        body,
        grid=(N // NB,),
        in_specs=[pl.BlockSpec((NB, F), lambda i: (i, 0)),
                  pl.BlockSpec((F, M), lambda i: (0, 0)),
                  pl.BlockSpec((NB, 1), lambda i: (i, 0))],
        out_specs=pl.BlockSpec((NB, M), lambda i: (i, 0)),
        out_shape=jax.ShapeDtypeStruct((N, M), jnp.float32),
    )(x, wcat, dinv)


def _tc_step(u, p, bkp2, g, H):
    def body(u_ref, p_ref, b_ref, g_ref, o_ref):
        o_ref[...] = (u_ref[...]
                      - 2.0 * g_ref[...] * (p_ref[0] + p_ref[1])
                      - b_ref[...])

    return pl.pallas_call(
        body,
        grid=(1,),
        in_specs=[pl.BlockSpec((N, H), lambda i: (0, 0)),
                  pl.BlockSpec((NC, N, H), lambda i: (0, 0, 0)),
                  pl.BlockSpec((N, H), lambda i: (0, 0)),
                  pl.BlockSpec((N, 1), lambda i: (0, 0))],
        out_specs=pl.BlockSpec((N, H), lambda i: (0, 0)),
        out_shape=jax.ShapeDtypeStruct((N, H), jnp.float32),
        )(u, p, bkp2, g)


def _tc_final(u0, alt, p, b2, dinv, dsqrt, bias, H, relu):
    def body(u0_ref, alt_ref, p_ref, b2_ref, dinv_ref, dsq_ref, bias_ref, o_ref):
        dinv_v = dinv_ref[...]
        main = u0_ref[...] - dinv_v * (p_ref[0] + p_ref[1]) - dsq_ref[...] * b2_ref[...]
        out = jnp.where(dinv_v > 0, main, alt_ref[...]) + bias_ref[...]
        if relu:
            out = jnp.maximum(out, 0.0)
        o_ref[...] = out

    return pl.pallas_call(
        body,
        grid=(1,),
        in_specs=[pl.BlockSpec((N, H), lambda i: (0, 0)),
                  pl.BlockSpec((N, H), lambda i: (0, 0)),
                  pl.BlockSpec((NC, N, H), lambda i: (0, 0, 0)),
                  pl.BlockSpec((N, H), lambda i: (0, 0)),
                  pl.BlockSpec((N, 1), lambda i: (0, 0)),
                  pl.BlockSpec((N, 1), lambda i: (0, 0)),
                  pl.BlockSpec((1, H), lambda i: (0, 0))],
        out_specs=pl.BlockSpec((N, H), lambda i: (0, 0)),
        out_shape=jax.ShapeDtypeStruct((N, H), jnp.float32),
    )(u0, alt, p, b2, dinv, dsqrt, bias)


_SC_DEGREE = _make_sc_degree()
_SC_PROP = {64: _make_sc_prop(NH1), 16: _make_sc_prop(H2)}


def _layer(x, W, bias, src2d, dst2d, dinv, g, dsqrt, H, relu):
    walt = W[0]
    for k in range(1, K):
        walt = walt + ((-1.0) ** k) * W[k]
    wcat = jnp.concatenate([W[k] for k in range(K)] + [walt], axis=1)

    U = _tc_project(x, wcat, dinv, H)
    Uk = [U[:, k * H:(k + 1) * H] for k in range(K + 1)]
    prop = _SC_PROP[H]
    zeros2 = jnp.zeros((N, H), jnp.float32)

    beta_kp1 = Uk[K - 1]
    beta_kp2 = zeros2
    for k in range(K - 2, 0, -1):
        p = prop(beta_kp1, src2d, dst2d, zeros2)
        newb = _tc_step(Uk[k], p, beta_kp2, g, H)
        beta_kp2, beta_kp1 = beta_kp1, newb
    p = prop(beta_kp1, src2d, dst2d, zeros2)
    return _tc_final(Uk[0], Uk[K], p, beta_kp2, dinv, dsqrt,
                     bias.reshape(1, H), H, relu)


def kernel(x, edge_index, W1, b1, W2, b2):
    src2d = edge_index[0].reshape(NW, ROWS_PW, C)
    dst2d = edge_index[1].reshape(NW, ROWS_PW, C)

    degp = _SC_DEGREE(src2d, jnp.zeros((N,), jnp.float32)).reshape(NC, N)
    dinv, g, dsqrt = _tc_precompute(degp[0].reshape(N, 1), degp[1].reshape(N, 1))

    h = _layer(x, W1, b1, src2d, dst2d, dinv, g, dsqrt, NH1, relu=True)

    W2p = jnp.pad(W2, ((0, 0), (0, 0), (0, H2 - NCLS)))
    b2p = jnp.pad(b2, (0, H2 - NCLS))
    out = _layer(h, W2p, b2p, src2d, dst2d, dinv, g, dsqrt, H2, relu=False)
    return out[:, :NCLS]


# speedup vs baseline: 19.3890x; 1.1823x over previous
import functools

import jax
import jax.numpy as jnp
from jax import lax
from jax.experimental import pallas as pl
from jax.experimental.pallas import tpu as pltpu
from jax.experimental.pallas import tpu_sc as plsc

N = 10000
E = 320000
F_IN = 128
NH1 = 64
NCLS = 10
K = 8
H2 = 16

NC = 2
NS = 16
NW = NC * NS

C = 80
ROWS_ALL = E // C
ROWS_PW = ROWS_ALL // NW
RPT = N // NS

_MESH = plsc.VectorSubcoreMesh(core_axis_name="c", subcore_axis_name="s")


RA = 632
RB = N - (NS - 1) * RA

_SC_PARAMS = pltpu.CompilerParams(use_tc_tiling_on_sc=False)


def _make_sc_degree():
    @functools.partial(
        pl.kernel,
        out_type=jax.ShapeDtypeStruct((NC * N,), jnp.float32),
        mesh=_MESH,
        compiler_params=_SC_PARAMS,
        scratch_types=[
            pltpu.VMEM((ROWS_PW, C), jnp.int32),
            pltpu.VMEM((C,), jnp.float32),
            pltpu.VMEM_SHARED((N,), jnp.float32),
        ],
    )
    def degree(srcr, zeros1, out, src_v, ones_v, dacc):
        cid = lax.axis_index("c")
        sid = lax.axis_index("s")
        wid = cid * NS + sid
        start = sid * RA

        @pl.loop(0, C, step=16)
        def _(j):
            ones_v[pl.ds(j, 16)] = jnp.ones((16,), jnp.float32)

        @pl.when(sid < NS - 1)
        def _():
            pltpu.sync_copy(zeros1.at[pl.ds(start, RA)], dacc.at[pl.ds(start, RA)])

        @pl.when(sid == NS - 1)
        def _():
            pltpu.sync_copy(zeros1.at[pl.ds(start, RB)], dacc.at[pl.ds(start, RB)])

        pltpu.sync_copy(srcr.at[wid], src_v)
        plsc.subcore_barrier()

        @pl.loop(0, ROWS_PW)
        def _(i):
            pltpu.sync_copy(ones_v, dacc.at[src_v.at[i]], add=True)

        plsc.subcore_barrier()

        @pl.when(sid < NS - 1)
        def _():
            pltpu.sync_copy(dacc.at[pl.ds(start, RA)],
                            out.at[pl.ds(cid * N + start, RA)])

        @pl.when(sid == NS - 1)
        def _():
            pltpu.sync_copy(dacc.at[pl.ds(start, RB)],
                            out.at[pl.ds(cid * N + start, RB)])

    return degree


def _make_sc_prop(H):
    @functools.partial(
        pl.kernel,
        out_type=jax.ShapeDtypeStruct((NC, N, H), jnp.float32),
        mesh=_MESH,
        compiler_params=_SC_PARAMS,
        scratch_types=[
            pltpu.VMEM((ROWS_PW, C), jnp.int32),
            pltpu.VMEM((ROWS_PW, C), jnp.int32),
            pltpu.VMEM((C, H), jnp.float32),
            pltpu.VMEM((C, H), jnp.float32),
            pltpu.VMEM_SHARED((N, H), jnp.float32),
            pltpu.VMEM_SHARED((N, H), jnp.float32),
            pltpu.SemaphoreType.DMA,
            pltpu.SemaphoreType.DMA,
        ],
    )
    def prop(beta, srcr, dstr, zeros2, out, src_v, dst_v, rows0, rows1, acc,
             bloc, s0, s1):
        cid = lax.axis_index("c")
        sid = lax.axis_index("s")
        wid = cid * NS + sid
        start = sid * RA

        @pl.when(sid < NS - 1)
        def _():
            pltpu.sync_copy(zeros2.at[pl.ds(start, RA)], acc.at[pl.ds(start, RA)])
            pltpu.sync_copy(beta.at[pl.ds(start, RA)], bloc.at[pl.ds(start, RA)])

        @pl.when(sid == NS - 1)
        def _():
            pltpu.sync_copy(zeros2.at[pl.ds(start, RB)], acc.at[pl.ds(start, RB)])
            pltpu.sync_copy(beta.at[pl.ds(start, RB)], bloc.at[pl.ds(start, RB)])

        pltpu.sync_copy(srcr.at[wid], src_v)
        pltpu.sync_copy(dstr.at[wid], dst_v)
        plsc.subcore_barrier()

        def gather(i, buf, sem):
            return pltpu.make_async_copy(bloc.at[src_v.at[i]], buf, sem)

        def scatter(i, buf):
            pltpu.sync_copy(buf, acc.at[dst_v.at[i]], add=True)

        gather(0, rows0, s0).start()

        @pl.loop(0, ROWS_PW - 1, step=2)
        def _(i):
            gather(i + 1, rows1, s1).start()
            gather(i, rows0, s0).wait()
            scatter(i, rows0)
            gather(i + 2, rows0, s0).start()
            gather(i + 1, rows1, s1).wait()
            scatter(i + 1, rows1)

        gather(ROWS_PW - 1, rows0, s0).wait()
        scatter(ROWS_PW - 1, rows0)

        plsc.subcore_barrier()

        @pl.when(sid < NS - 1)
        def _():
            pltpu.sync_copy(acc.at[pl.ds(start, RA)],
                            out.at[cid, pl.ds(start, RA)])

        @pl.when(sid == NS - 1)
        def _():
            pltpu.sync_copy(acc.at[pl.ds(start, RB)],
                            out.at[cid, pl.ds(start, RB)])

    return prop


def _tc_precompute(deg0, deg1):
    def body(d0_ref, d1_ref, dinv_ref, g_ref, dsq_ref):
        deg = d0_ref[...] + d1_ref[...]
        pos = deg > 0
        safe = jnp.where(pos, deg, 1.0)
        dinv_ref[...] = jnp.where(pos, lax.rsqrt(safe), 0.0)
        g_ref[...] = jnp.where(pos, 1.0 / safe, 0.0)
        dsq_ref[...] = jnp.where(pos, jnp.sqrt(safe), 0.0)

    shp = jax.ShapeDtypeStruct((N, 1), jnp.float32)
    return pl.pallas_call(
        body,
        grid=(1,),
        in_specs=[pl.BlockSpec((N, 1), lambda i: (0, 0)),
                  pl.BlockSpec((N, 1), lambda i: (0, 0))],
        out_specs=[pl.BlockSpec((N, 1), lambda i: (0, 0))] * 3,
        out_shape=[shp, shp, shp],
    )(deg0, deg1)


def _tc_project(x, wcat, dinv, H):
    F = x.shape[1]
    M = wcat.shape[1]
    NB = 1000

    def body(x_ref, w_ref, d_ref, o_ref):
        u = jnp.dot(x_ref[...], w_ref[...], preferred_element_type=jnp.float32)
        col = lax.broadcasted_iota(jnp.int32, (NB, M), 1)
        m = (col >= H) & (col < 8 * H)
        o_ref[...] = jnp.where(m, u * d_ref[...], u)

    return pl.pallas_call(
        body,
        grid=(N // NB,),
        in_specs=[pl.BlockSpec((NB, F), lambda i: (i, 0)),
                  pl.BlockSpec((F, M), lambda i: (0, 0)),
                  pl.BlockSpec((NB, 1), lambda i: (i, 0))],
        out_specs=pl.BlockSpec((NB, M), lambda i: (i, 0)),
        out_shape=jax.ShapeDtypeStruct((N, M), jnp.float32),
    )(x, wcat, dinv)


def _tc_step(u, p, bkp2, g, H):
    def body(u_ref, p_ref, b_ref, g_ref, o_ref):
        o_ref[...] = (u_ref[...]
                      - 2.0 * g_ref[...] * (p_ref[0] + p_ref[1])
                      - b_ref[...])

    return pl.pallas_call(
        body,
        grid=(1,),
        in_specs=[pl.BlockSpec((N, H), lambda i: (0, 0)),
                  pl.BlockSpec((NC, N, H), lambda i: (0, 0, 0)),
                  pl.BlockSpec((N, H), lambda i: (0, 0)),
                  pl.BlockSpec((N, 1), lambda i: (0, 0))],
        out_specs=pl.BlockSpec((N, H), lambda i: (0, 0)),
        out_shape=jax.ShapeDtypeStruct((N, H), jnp.float32),
        )(u, p, bkp2, g)


def _tc_final(u0, alt, p, b2, dinv, dsqrt, bias, H, relu):
    def body(u0_ref, alt_ref, p_ref, b2_ref, dinv_ref, dsq_ref, bias_ref, o_ref):
        dinv_v = dinv_ref[...]
        main = u0_ref[...] - dinv_v * (p_ref[0] + p_ref[1]) - dsq_ref[...] * b2_ref[...]
        out = jnp.where(dinv_v > 0, main, alt_ref[...]) + bias_ref[...]
        if relu:
            out = jnp.maximum(out, 0.0)
        o_ref[...] = out

    return pl.pallas_call(
        body,
        grid=(1,),
        in_specs=[pl.BlockSpec((N, H), lambda i: (0, 0)),
                  pl.BlockSpec((N, H), lambda i: (0, 0)),
                  pl.BlockSpec((NC, N, H), lambda i: (0, 0, 0)),
                  pl.BlockSpec((N, H), lambda i: (0, 0)),
                  pl.BlockSpec((N, 1), lambda i: (0, 0)),
                  pl.BlockSpec((N, 1), lambda i: (0, 0)),
                  pl.BlockSpec((1, H), lambda i: (0, 0))],
        out_specs=pl.BlockSpec((N, H), lambda i: (0, 0)),
        out_shape=jax.ShapeDtypeStruct((N, H), jnp.float32),
    )(u0, alt, p, b2, dinv, dsqrt, bias)


_SC_DEGREE = _make_sc_degree()
_SC_PROP = {64: _make_sc_prop(NH1), 16: _make_sc_prop(H2)}


def _layer(x, W, bias, src2d, dst2d, dinv, g, dsqrt, H, relu):
    walt = W[0]
    for k in range(1, K):
        walt = walt + ((-1.0) ** k) * W[k]
    wcat = jnp.concatenate([W[k] for k in range(K)] + [walt], axis=1)

    U = _tc_project(x, wcat, dinv, H)
    Uk = [U[:, k * H:(k + 1) * H] for k in range(K + 1)]
    prop = _SC_PROP[H]
    zeros2 = jnp.zeros((N, H), jnp.float32)

    beta_kp1 = Uk[K - 1]
    beta_kp2 = zeros2
    for k in range(K - 2, 0, -1):
        p = prop(beta_kp1, src2d, dst2d, zeros2)
        newb = _tc_step(Uk[k], p, beta_kp2, g, H)
        beta_kp2, beta_kp1 = beta_kp1, newb
    p = prop(beta_kp1, src2d, dst2d, zeros2)
    return _tc_final(Uk[0], Uk[K], p, beta_kp2, dinv, dsqrt,
                     bias.reshape(1, H), H, relu)


def kernel(x, edge_index, W1, b1, W2, b2):
    src2d = edge_index[0].reshape(NW, ROWS_PW, C)
    dst2d = edge_index[1].reshape(NW, ROWS_PW, C)

    degp = _SC_DEGREE(src2d, jnp.zeros((N,), jnp.float32)).reshape(NC, N)
    dinv, g, dsqrt = _tc_precompute(degp[0].reshape(N, 1), degp[1].reshape(N, 1))

    h = _layer(x, W1, b1, src2d, dst2d, dinv, g, dsqrt, NH1, relu=True)

    W2p = jnp.pad(W2, ((0, 0), (0, 0), (0, H2 - NCLS)))
    b2p = jnp.pad(b2, (0, H2 - NCLS))
    out = _layer(h, W2p, b2p, src2d, dst2d, dinv, g, dsqrt, H2, relu=False)
    return out[:, :NCLS]
